# baseline (speedup 1.0000x reference)
import functools

import jax
import jax.numpy as jnp
from jax import lax
from jax.experimental import pallas as pl
from jax.experimental.pallas import tpu as pltpu
from jax.experimental.pallas import tpu_sc as plsc

_NC = 2
_NS = 16
_NW = _NC * _NS
_L = 16

_T = 100
_T_PAD = 112
_IN_SIZE = 512.0

_mesh = plsc.VectorSubcoreMesh(core_axis_name="c", subcore_axis_name="s")


def _make_sc_kernel(npw, imgs_per_worker_div):
    @functools.partial(
        pl.kernel,
        out_type=jax.ShapeDtypeStruct((_NW, npw), jnp.float32),
        mesh=_mesh,
        compiler_params=pltpu.CompilerParams(needs_layout_passes=False),
        scratch_types=[
            pltpu.VMEM((4, npw), jnp.float32),
            pltpu.VMEM((4, _T_PAD), jnp.float32),
            pltpu.VMEM((_T_PAD,), jnp.float32),
            pltpu.VMEM((_T_PAD,), jnp.float32),
            pltpu.VMEM((_T_PAD,), jnp.float32),
            pltpu.VMEM((_T_PAD,), jnp.float32),
            pltpu.VMEM((_T_PAD,), jnp.float32),
            pltpu.VMEM((npw,), jnp.float32),
            pltpu.VMEM((npw,), jnp.float32),
            pltpu.VMEM((npw,), jnp.float32),
        ],
    )
    def sc_kernel(pred_hbm, tgt_hbm, mask_hbm, out_hbm,
                  pred_v, tgt_v, tx1_v, ty1_v, tx2_v, ty2_v, ta_v,
                  macc_v, mask_v, out_v):
        wid = lax.axis_index("s") * _NC + lax.axis_index("c")
        img = wid // imgs_per_worker_div

        pltpu.sync_copy(pred_hbm.at[wid], pred_v)
        pltpu.sync_copy(tgt_hbm.at[img], tgt_v)
        pltpu.sync_copy(mask_hbm.at[wid], mask_v)

        def tprep(j, carry):
            sl = pl.ds(j * _L, _L)
            cx = tgt_v[0, sl] * _IN_SIZE
            cy = tgt_v[1, sl] * _IN_SIZE
            hw = tgt_v[2, sl] * (0.5 * _IN_SIZE)
            hh = tgt_v[3, sl] * (0.5 * _IN_SIZE)
            x1 = cx - hw
            y1 = cy - hh
            x2 = cx + hw
            y2 = cy + hh
            tx1_v[sl] = x1
            ty1_v[sl] = y1
            tx2_v[sl] = x2
            ty2_v[sl] = y2
            ta_v[sl] = (x2 - x1) * (y2 - y1)
            return carry

        lax.fori_loop(0, _T_PAD // _L, tprep, 0)

        def pprep(p, carry):
            sl = pl.ds(p * _L, _L)
            cx = pred_v[0, sl]
            cy = pred_v[1, sl]
            hw = pred_v[2, sl] * 0.5
            hh = pred_v[3, sl] * 0.5
            pred_v[0, sl] = cx - hw
            pred_v[1, sl] = cy - hh
            pred_v[2, sl] = cx + hw
            pred_v[3, sl] = cy + hh
            macc_v[sl] = jnp.zeros((_L,), jnp.float32)
            return carry

        lax.fori_loop(0, npw // _L, pprep, 0)

        def tloop(t, carry):
            idx = jnp.full((_L,), t, jnp.int32)
            btx1 = plsc.load_gather(tx1_v, [idx])
            bty1 = plsc.load_gather(ty1_v, [idx])
            btx2 = plsc.load_gather(tx2_v, [idx])
            bty2 = plsc.load_gather(ty2_v, [idx])
            bta = plsc.load_gather(ta_v, [idx])

            def ploop(p, c2):
                sl = pl.ds(p * _L, _L)
                iw = jnp.maximum(
                    jnp.minimum(btx2, pred_v[2, sl])
                    - jnp.maximum(btx1, pred_v[0, sl]), 0.0)
                ih = jnp.maximum(
                    jnp.minimum(bty2, pred_v[3, sl])
                    - jnp.maximum(bty1, pred_v[1, sl]), 0.0)
                s = iw * ih * 3.0 - bta
                macc_v[sl] = jnp.maximum(macc_v[sl], s)
                return c2

            lax.fori_loop(0, npw // _L, ploop, 0)
            return carry

        lax.fori_loop(0, _T, tloop, 0)

        def fin(p, carry):
            sl = pl.ds(p * _L, _L)
            pa = (pred_v[2, sl] - pred_v[0, sl]) * (pred_v[3, sl] - pred_v[1, sl])
            out_v[sl] = jnp.where(macc_v[sl] > pa, 0.0, mask_v[sl])
            return carry

        lax.fori_loop(0, npw // _L, fin, 0)
        pltpu.sync_copy(out_v, out_hbm.at[wid])

    return sc_kernel


def kernel(batch_predict_boxes, batch_targets, no_obj_mask):
    b = batch_predict_boxes.shape[0]
    n = 1
    for d in batch_predict_boxes.shape[1:-1]:
        n *= d
    total = b * n
    npw = total // _NW
    div = _NW // b

    pred = batch_predict_boxes.reshape(_NW, npw, 4).transpose(0, 2, 1)
    tgt = jnp.pad(batch_targets, ((0, 0), (0, _T_PAD - _T), (0, 0)))
    tgt = tgt.transpose(0, 2, 1)
    mask = no_obj_mask.reshape(_NW, npw)

    out = _make_sc_kernel(npw, div)(pred, tgt, mask)
    return out.reshape(no_obj_mask.shape)


# speedup vs baseline: 3.9223x; 3.9223x over previous
import functools

import jax
import jax.numpy as jnp
from jax import lax
from jax.experimental import pallas as pl
from jax.experimental.pallas import tpu as pltpu
from jax.experimental.pallas import tpu_sc as plsc

_NC = 2
_NS = 16
_NW = _NC * _NS
_L = 16

_T = 100
_T_PAD = 112
_TK = 4
_IN_SIZE = 512.0

_mesh = plsc.VectorSubcoreMesh(core_axis_name="c", subcore_axis_name="s")


def _make_sc_kernel(npw, imgs_per_worker_div):
    @functools.partial(
        pl.kernel,
        out_type=jax.ShapeDtypeStruct((_NW, npw), jnp.float32),
        mesh=_mesh,
        compiler_params=pltpu.CompilerParams(needs_layout_passes=False),
        scratch_types=[
            pltpu.VMEM((4, npw), jnp.float32),
            pltpu.VMEM((4, _T_PAD), jnp.float32),
            pltpu.VMEM((_T_PAD,), jnp.float32),
            pltpu.VMEM((_T_PAD,), jnp.float32),
            pltpu.VMEM((_T_PAD,), jnp.float32),
            pltpu.VMEM((_T_PAD,), jnp.float32),
            pltpu.VMEM((_T_PAD,), jnp.float32),
            pltpu.VMEM((npw,), jnp.float32),
            pltpu.VMEM((npw,), jnp.float32),
            pltpu.VMEM((npw,), jnp.float32),
        ],
    )
    def sc_kernel(pred_hbm, tgt_hbm, mask_hbm, out_hbm,
                  pred_v, tgt_v, tx1_v, ty1_v, tx2_v, ty2_v, ta3_v,
                  macc_v, mask_v, out_v):
        wid = lax.axis_index("s") * _NC + lax.axis_index("c")
        img = wid // imgs_per_worker_div

        pltpu.sync_copy(pred_hbm.at[wid], pred_v)
        pltpu.sync_copy(tgt_hbm.at[img], tgt_v)
        pltpu.sync_copy(mask_hbm.at[wid], mask_v)

        for j in range(_T_PAD // _L):
            sl = pl.ds(j * _L, _L)
            cx = tgt_v[0, sl] * _IN_SIZE
            cy = tgt_v[1, sl] * _IN_SIZE
            hw = tgt_v[2, sl] * (0.5 * _IN_SIZE)
            hh = tgt_v[3, sl] * (0.5 * _IN_SIZE)
            x1 = cx - hw
            y1 = cy - hh
            x2 = cx + hw
            y2 = cy + hh
            tx1_v[sl] = x1
            ty1_v[sl] = y1
            tx2_v[sl] = x2
            ty2_v[sl] = y2
            ta3_v[sl] = (x2 - x1) * (y2 - y1) * (1.0 / 3.0)

        @plsc.parallel_loop(0, npw // _L, 1, unroll=4)
        def pprep(p):
            sl = pl.ds(p * _L, _L)
            cx = pred_v[0, sl]
            cy = pred_v[1, sl]
            hw = pred_v[2, sl] * 0.5
            hh = pred_v[3, sl] * 0.5
            pred_v[0, sl] = cx - hw
            pred_v[1, sl] = cy - hh
            pred_v[2, sl] = cx + hw
            pred_v[3, sl] = cy + hh
            macc_v[sl] = jnp.zeros((_L,), jnp.float32)

        def tblk(tb, carry):
            t0 = tb * _TK
            bts = []
            for k in range(_TK):
                idx = jnp.full((_L,), t0 + k, jnp.int32)
                bts.append((plsc.load_gather(tx1_v, [idx]),
                            plsc.load_gather(ty1_v, [idx]),
                            plsc.load_gather(tx2_v, [idx]),
                            plsc.load_gather(ty2_v, [idx]),
                            plsc.load_gather(ta3_v, [idx])))

            @plsc.parallel_loop(0, npw // _L, 1, unroll=4)
            def ploop(p):
                sl = pl.ds(p * _L, _L)
                px1 = pred_v[0, sl]
                py1 = pred_v[1, sl]
                px2 = pred_v[2, sl]
                py2 = pred_v[3, sl]
                m = macc_v[sl]
                for (btx1, bty1, btx2, bty2, bta3) in bts:
                    iw = jnp.maximum(
                        jnp.minimum(btx2, px2) - jnp.maximum(btx1, px1), 0.0)
                    ih = jnp.maximum(
                        jnp.minimum(bty2, py2) - jnp.maximum(bty1, py1), 0.0)
                    m = jnp.maximum(m, iw * ih - bta3)
                macc_v[sl] = m

            return carry

        lax.fori_loop(0, _T // _TK, tblk, 0)

        @plsc.parallel_loop(0, npw // _L, 1, unroll=4)
        def fin(p):
            sl = pl.ds(p * _L, _L)
            pa3 = ((pred_v[2, sl] - pred_v[0, sl])
                   * (pred_v[3, sl] - pred_v[1, sl]) * (1.0 / 3.0))
            out_v[sl] = jnp.where(macc_v[sl] > pa3, 0.0, mask_v[sl])

        pltpu.sync_copy(out_v, out_hbm.at[wid])

    return sc_kernel


def kernel(batch_predict_boxes, batch_targets, no_obj_mask):
    b = batch_predict_boxes.shape[0]
    n = 1
    for d in batch_predict_boxes.shape[1:-1]:
        n *= d
    total = b * n
    npw = total // _NW
    div = _NW // b

    pred = batch_predict_boxes.reshape(_NW, npw, 4).transpose(0, 2, 1)
    tgt = jnp.pad(batch_targets, ((0, 0), (0, _T_PAD - _T), (0, 0)))
    tgt = tgt.transpose(0, 2, 1)
    mask = no_obj_mask.reshape(_NW, npw)

    out = _make_sc_kernel(npw, div)(pred, tgt, mask)
    return out.reshape(no_obj_mask.shape)


# speedup vs baseline: 10.5391x; 2.6870x over previous
import functools

import jax
import jax.numpy as jnp
from jax import lax
from jax.experimental import pallas as pl
from jax.experimental.pallas import tpu as pltpu
from jax.experimental.pallas import tpu_sc as plsc

_NC = 2
_NS = 16
_NW = _NC * _NS
_L = 16

_T = 100
_T_PAD = 112
_TK = 4
_IN_SIZE = 512.0

_mesh = plsc.VectorSubcoreMesh(core_axis_name="c", subcore_axis_name="s")


def _make_sc_kernel(npw, imgs_per_worker_div):
    @functools.partial(
        pl.kernel,
        out_type=jax.ShapeDtypeStruct((_NW, npw), jnp.float32),
        mesh=_mesh,
        compiler_params=pltpu.CompilerParams(needs_layout_passes=False),
        scratch_types=[
            pltpu.VMEM((4, npw), jnp.float32),
            pltpu.VMEM((4, _T_PAD), jnp.float32),
            pltpu.VMEM((_T_PAD,), jnp.float32),
            pltpu.VMEM((_T_PAD,), jnp.float32),
            pltpu.VMEM((_T_PAD,), jnp.float32),
            pltpu.VMEM((_T_PAD,), jnp.float32),
            pltpu.VMEM((_T_PAD,), jnp.float32),
            pltpu.VMEM((npw,), jnp.float32),
            pltpu.VMEM((npw,), jnp.float32),
            pltpu.VMEM((npw,), jnp.float32),
        ],
    )
    def sc_kernel(pred_hbm, tgt_hbm, mask_hbm, out_hbm,
                  pred_v, tgt_v, tx1_v, ty1_v, tx2_v, ty2_v, ta3_v,
                  macc_v, mask_v, out_v):
        wid = lax.axis_index("s") * _NC + lax.axis_index("c")
        img = wid // imgs_per_worker_div

        pltpu.sync_copy(pred_hbm.at[wid], pred_v)
        pltpu.sync_copy(tgt_hbm.at[img], tgt_v)
        pltpu.sync_copy(mask_hbm.at[wid], mask_v)

        for j in range(_T_PAD // _L):
            sl = pl.ds(j * _L, _L)
            cx = tgt_v[0, sl] * _IN_SIZE
            cy = tgt_v[1, sl] * _IN_SIZE
            hw = tgt_v[2, sl] * (0.5 * _IN_SIZE)
            hh = tgt_v[3, sl] * (0.5 * _IN_SIZE)
            x1 = cx - hw
            y1 = cy - hh
            x2 = cx + hw
            y2 = cy + hh
            tx1_v[sl] = x1
            ty1_v[sl] = y1
            tx2_v[sl] = x2
            ty2_v[sl] = y2
            ta3_v[sl] = (x2 - x1) * (y2 - y1) * (1.0 / 3.0)

        init = (jnp.full((_L,), -1.0, jnp.float32),
                jnp.full((_L,), 3.4e38, jnp.float32))

        @plsc.parallel_loop(0, npw // _L, 1, unroll=4, carry=init)
        def pprep(p, ext):
            pamax3, pamin3 = ext
            sl = pl.ds(p * _L, _L)
            cx = pred_v[0, sl]
            cy = pred_v[1, sl]
            hw = pred_v[2, sl] * 0.5
            hh = pred_v[3, sl] * 0.5
            x1 = cx - hw
            y1 = cy - hh
            x2 = cx + hw
            y2 = cy + hh
            pred_v[0, sl] = x1
            pred_v[1, sl] = y1
            pred_v[2, sl] = x2
            pred_v[3, sl] = y2
            macc_v[sl] = jnp.zeros((_L,), jnp.float32)
            pa3 = (x2 - x1) * (y2 - y1) * (1.0 / 3.0)
            return jnp.maximum(pamax3, pa3), jnp.minimum(pamin3, pa3)

        pamax3 = jnp.max(pprep[0])
        pamin3 = jnp.min(pprep[1])

        def tblk(tb, carry):
            t0 = tb * _TK
            bts = []
            for k in range(_TK):
                idx = jnp.full((_L,), t0 + k, jnp.int32)
                bts.append((plsc.load_gather(tx1_v, [idx]),
                            plsc.load_gather(ty1_v, [idx]),
                            plsc.load_gather(tx2_v, [idx]),
                            plsc.load_gather(ty2_v, [idx]),
                            plsc.load_gather(ta3_v, [idx])))

            alive = jnp.zeros((_L,), jnp.int32)
            for (_, _, _, _, bta3) in bts:
                live = jnp.logical_and(bta3 < pamax3 * 5.0,
                                       bta3 * 5.0 > pamin3)
                alive = alive | jnp.where(live, 1, 0)

            @pl.when(jnp.max(alive) > 0)
            def _():
                @plsc.parallel_loop(0, npw // _L, 1, unroll=4)
                def ploop(p):
                    sl = pl.ds(p * _L, _L)
                    px1 = pred_v[0, sl]
                    py1 = pred_v[1, sl]
                    px2 = pred_v[2, sl]
                    py2 = pred_v[3, sl]
                    m = macc_v[sl]
                    for (btx1, bty1, btx2, bty2, bta3) in bts:
                        iw = jnp.maximum(
                            jnp.minimum(btx2, px2) - jnp.maximum(btx1, px1),
                            0.0)
                        ih = jnp.maximum(
                            jnp.minimum(bty2, py2) - jnp.maximum(bty1, py1),
                            0.0)
                        m = jnp.maximum(m, iw * ih - bta3)
                    macc_v[sl] = m

            return carry

        lax.fori_loop(0, _T // _TK, tblk, 0)

        @plsc.parallel_loop(0, npw // _L, 1, unroll=4)
        def fin(p):
            sl = pl.ds(p * _L, _L)
            pa3 = ((pred_v[2, sl] - pred_v[0, sl])
                   * (pred_v[3, sl] - pred_v[1, sl]) * (1.0 / 3.0))
            out_v[sl] = jnp.where(macc_v[sl] > pa3, 0.0, mask_v[sl])

        pltpu.sync_copy(out_v, out_hbm.at[wid])

    return sc_kernel


def kernel(batch_predict_boxes, batch_targets, no_obj_mask):
    b = batch_predict_boxes.shape[0]
    n = 1
    for d in batch_predict_boxes.shape[1:-1]:
        n *= d
    total = b * n
    npw = total // _NW
    div = _NW // b

    pred = batch_predict_boxes.reshape(_NW, npw, 4).transpose(0, 2, 1)
    tgt = jnp.pad(batch_targets, ((0, 0), (0, _T_PAD - _T), (0, 0)))
    tgt = tgt.transpose(0, 2, 1)
    mask = no_obj_mask.reshape(_NW, npw)

    out = _make_sc_kernel(npw, div)(pred, tgt, mask)
    return out.reshape(no_obj_mask.shape)
